# Initial kernel scaffold; baseline (speedup 1.0000x reference)
import jax
import jax.numpy as jnp
from jax.experimental import pallas as pl


def kernel(x, target):
    raise NotImplementedError("write your pallas kernel here")


# speedup vs baseline: 3.0475x; 3.0475x over previous
import functools

import jax
import jax.numpy as jnp
from jax.experimental import pallas as pl
from jax.experimental.pallas import tpu as pltpu

_BINS = 10
_LANES = 128


def _ghmc_kernel(x_ref, t_ref, out_ref, cnt_ref, s_ref, *, nblocks, n_rows):
    i = pl.program_id(0)

    @pl.when(i == 0)
    def _init():
        cnt_ref[...] = jnp.zeros_like(cnt_ref)
        s_ref[...] = jnp.zeros_like(s_ref)

    xb = x_ref[...]
    t = t_ref[...]
    m = jnp.max(xb, axis=1, keepdims=True)
    se = jnp.sum(jnp.exp(xb - m), axis=1, keepdims=True)
    lse = m + jnp.log(se)
    col = jax.lax.broadcasted_iota(jnp.int32, xb.shape, 1)
    xt = jnp.sum(jnp.where(col == t, xb, 0.0), axis=1, keepdims=True)
    log_pt = xt - lse
    ce = -log_pt
    g = jnp.abs(jnp.exp(log_pt) - 1.0)
    bin_idx = jnp.floor(g * (_BINS - 0.0001)).astype(jnp.int32)

    lane = jax.lax.broadcasted_iota(jnp.int32, (xb.shape[0], _LANES), 1)
    onehot = lane == bin_idx
    cnt_ref[...] += jnp.sum(onehot.astype(jnp.float32), axis=0, keepdims=True)
    s_ref[...] += jnp.sum(jnp.where(onehot, ce, 0.0), axis=0, keepdims=True)

    @pl.when(i == nblocks - 1)
    def _finish():
        cnt = cnt_ref[...]
        lane2 = jax.lax.broadcasted_iota(jnp.int32, cnt.shape, 1)
        valid = lane2 < _BINS
        nonempty = jnp.sum(jnp.where(valid & (cnt > 0.0), 1.0, 0.0),
                           keepdims=True)
        gd = jnp.maximum(cnt * nonempty, 0.0001)
        beta = 1.0 / gd
        loss = jnp.sum(jnp.where(valid, beta * s_ref[...], 0.0),
                       keepdims=True) / n_rows
        out_ref[...] = loss


def kernel(x, target):
    n, c = x.shape
    block_rows = 1024
    nblocks = n // block_rows
    t2 = target.reshape(n, 1)

    out = pl.pallas_call(
        functools.partial(_ghmc_kernel, nblocks=nblocks, n_rows=n),
        grid=(nblocks,),
        in_specs=[
            pl.BlockSpec((block_rows, c), lambda i: (i, 0)),
            pl.BlockSpec((block_rows, 1), lambda i: (i, 0)),
        ],
        out_specs=pl.BlockSpec((1, 1), lambda i: (0, 0)),
        out_shape=jax.ShapeDtypeStruct((1, 1), jnp.float32),
        scratch_shapes=[
            pltpu.VMEM((1, _LANES), jnp.float32),
            pltpu.VMEM((1, _LANES), jnp.float32),
        ],
    )(x, t2)
    return out[0, 0]
